# baseline (speedup 1.0000x reference)
import functools

import jax
import jax.numpy as jnp
from jax.experimental import pallas as pl
from jax.experimental.pallas import tpu as pltpu

_EPS = 1e-5


def _fused_mlp_kernel(x_ref, W1_ref, g1_ref, be1_ref, W2_ref, g2_ref, be2_ref,
                      Wf1_ref, bf1_ref, Wf2_ref, bf2_ref, out_ref, h_scr,
                      *, n_rows, bm):
    nchunks = n_rows // bm
    inv_n = 1.0 / n_rows
    W1 = W1_ref[...]

    x = x_ref[...]
    xt = jnp.transpose(x)
    S = jnp.dot(xt, x, preferred_element_type=jnp.float32)
    cs = jnp.sum(x, axis=0, keepdims=True)
    mean1 = jnp.sum(cs.reshape(-1, 1) * W1, axis=0, keepdims=True) * inv_n
    T = jnp.dot(S, W1, preferred_element_type=jnp.float32)
    m2 = jnp.sum(W1 * T, axis=0, keepdims=True) * inv_n
    var1 = m2 - mean1 * mean1
    sc1 = g1_ref[...].reshape(1, -1) * jax.lax.rsqrt(var1 + _EPS)
    sh1 = be1_ref[...].reshape(1, -1) - mean1 * sc1
    W1s = W1 * sc1

    W2 = W2_ref[...]
    s = jnp.zeros((1, W2.shape[1]), jnp.float32)
    q = jnp.zeros((1, W2.shape[1]), jnp.float32)
    for k in range(nchunks):
        rows = pl.ds(k * bm, bm)
        u = jnp.dot(x_ref[rows, :], W1s, preferred_element_type=jnp.float32)
        h1 = jnp.maximum(u + sh1, 0.0)
        h2 = jnp.dot(h1, W2, preferred_element_type=jnp.float32)
        h_scr[rows, :] = h2
        s = s + jnp.sum(h2, axis=0, keepdims=True)
        q = q + jnp.sum(h2 * h2, axis=0, keepdims=True)

    mean2 = s * inv_n
    var2 = q * inv_n - mean2 * mean2
    sc2 = g2_ref[...].reshape(1, -1) * jax.lax.rsqrt(var2 + _EPS)
    sh2 = be2_ref[...].reshape(1, -1) - mean2 * sc2
    Wf1s = Wf1_ref[...] * sc2.reshape(-1, 1)
    c = (jnp.sum(sh2.reshape(-1, 1) * Wf1_ref[...], axis=0, keepdims=True)
         + bf1_ref[...].reshape(1, -1))

    Wf2 = Wf2_ref[...]
    out_c = Wf2.shape[1]
    Wf2p = jnp.concatenate(
        [Wf2, jnp.zeros((Wf2.shape[0], 128 - out_c), jnp.float32)], axis=1)
    bf2 = bf2_ref[...].reshape(1, -1)
    for k in range(nchunks):
        rows = pl.ds(k * bm, bm)
        m = jnp.dot(h_scr[rows, :], Wf1s, preferred_element_type=jnp.float32)
        m = jnp.maximum(m + c, 0.0)
        res = jnp.dot(m, Wf2p, preferred_element_type=jnp.float32)
        out_ref[rows, :] = res[:, :out_c] + bf2


def kernel(x, edge_index, edge_attr, W1, b1, g1, be1, W2, b2, g2, be2,
           Wf1, bf1, Wf2, bf2):
    del edge_index, edge_attr, b1, b2
    n, f_in = x.shape
    h_dim = W1.shape[1]
    out_c = Wf2.shape[1]

    body = functools.partial(_fused_mlp_kernel, n_rows=n, bm=1000)
    out = pl.pallas_call(
        body,
        out_shape=jax.ShapeDtypeStruct((n, out_c), jnp.float32),
        scratch_shapes=[
            pltpu.VMEM((n, h_dim), jnp.float32),
        ],
    )(x, W1, g1, be1, W2, g2, be2, Wf1, bf1, Wf2, bf2)
    return out


# speedup vs baseline: 1.0268x; 1.0268x over previous
import functools

import jax
import jax.numpy as jnp
from jax.experimental import pallas as pl
from jax.experimental.pallas import tpu as pltpu

_EPS = 1e-5


def _fused_mlp_kernel(x_hbm, W1_ref, g1_ref, be1_ref, W2_ref, g2_ref, be2_ref,
                      Wf1_ref, bf1_ref, Wf2_ref, bf2_ref, out_hbm,
                      h_scr, x_vmem, o_vmem, xsem, osem,
                      *, n_rows, bm):
    nchunks = n_rows // bm
    inv_n = 1.0 / n_rows
    f32 = jnp.float32

    def x_copy(k):
        rows = pl.ds(k * bm, bm)
        return pltpu.make_async_copy(x_hbm.at[rows, :], x_vmem.at[rows, :],
                                     xsem.at[k])

    def o_copy(k):
        rows = pl.ds(k * bm, bm)
        return pltpu.make_async_copy(o_vmem.at[rows, :], out_hbm.at[rows, :],
                                     osem.at[k])

    for k in range(nchunks):
        x_copy(k).start()

    W1 = W1_ref[...]
    S = jnp.zeros((W1.shape[0], W1.shape[0]), f32)
    cs = jnp.zeros((1, W1.shape[0]), f32)
    for k in range(nchunks):
        x_copy(k).wait()
        xk = x_vmem[pl.ds(k * bm, bm), :]
        S = S + jnp.dot(jnp.transpose(xk), xk, preferred_element_type=f32)
        cs = cs + jnp.sum(xk, axis=0, keepdims=True)

    mean1 = jnp.sum(cs.reshape(-1, 1) * W1, axis=0, keepdims=True) * inv_n
    T = jnp.dot(S, W1, preferred_element_type=f32)
    m2 = jnp.sum(W1 * T, axis=0, keepdims=True) * inv_n
    var1 = m2 - mean1 * mean1
    sc1 = g1_ref[...].reshape(1, -1) * jax.lax.rsqrt(var1 + _EPS)
    sh1 = be1_ref[...].reshape(1, -1) - mean1 * sc1
    W1s = W1 * sc1

    W2 = W2_ref[...]
    s = jnp.zeros((1, W2.shape[1]), f32)
    q = jnp.zeros((1, W2.shape[1]), f32)
    for k in range(nchunks):
        rows = pl.ds(k * bm, bm)
        u = jnp.dot(x_vmem[rows, :], W1s, preferred_element_type=f32)
        h1 = jnp.maximum(u + sh1, 0.0)
        h2 = jnp.dot(h1, W2, preferred_element_type=f32)
        h_scr[rows, :] = h2
        s = s + jnp.sum(h2, axis=0, keepdims=True)
        q = q + jnp.sum(h2 * h2, axis=0, keepdims=True)

    mean2 = s * inv_n
    var2 = q * inv_n - mean2 * mean2
    sc2 = g2_ref[...].reshape(1, -1) * jax.lax.rsqrt(var2 + _EPS)
    sh2 = be2_ref[...].reshape(1, -1) - mean2 * sc2
    Wf1s = Wf1_ref[...] * sc2.reshape(-1, 1)
    c = (jnp.sum(sh2.reshape(-1, 1) * Wf1_ref[...], axis=0, keepdims=True)
         + bf1_ref[...].reshape(1, -1))

    Wf2 = Wf2_ref[...]
    out_c = Wf2.shape[1]
    Wf2p = jnp.concatenate(
        [Wf2, jnp.zeros((Wf2.shape[0], 128 - out_c), f32)], axis=1)
    bf2 = bf2_ref[...].reshape(1, -1)
    for k in range(nchunks):
        rows = pl.ds(k * bm, bm)
        m = jnp.dot(h_scr[rows, :], Wf1s, preferred_element_type=f32)
        m = jnp.maximum(m + c, 0.0)
        res = jnp.dot(m, Wf2p, preferred_element_type=f32)
        o_vmem[rows, :] = res[:, :out_c] + bf2
        o_copy(k).start()
    for k in range(nchunks):
        o_copy(k).wait()


def kernel(x, edge_index, edge_attr, W1, b1, g1, be1, W2, b2, g2, be2,
           Wf1, bf1, Wf2, bf2):
    del edge_index, edge_attr, b1, b2
    n, f_in = x.shape
    h_dim = W1.shape[1]
    out_c = Wf2.shape[1]
    bm = 2000

    body = functools.partial(_fused_mlp_kernel, n_rows=n, bm=bm)
    out = pl.pallas_call(
        body,
        in_specs=[pl.BlockSpec(memory_space=pl.MemorySpace.ANY)] + [pl.BlockSpec()] * 10,
        out_specs=pl.BlockSpec(memory_space=pl.MemorySpace.ANY),
        out_shape=jax.ShapeDtypeStruct((n, out_c), jnp.float32),
        scratch_shapes=[
            pltpu.VMEM((n, h_dim), jnp.float32),
            pltpu.VMEM((n, f_in), jnp.float32),
            pltpu.VMEM((n, out_c), jnp.float32),
            pltpu.SemaphoreType.DMA((n // bm,)),
            pltpu.SemaphoreType.DMA((n // bm,)),
        ],
    )(x, W1, g1, be1, W2, g2, be2, Wf1, bf1, Wf2, bf2)
    return out


# speedup vs baseline: 1.0271x; 1.0003x over previous
import functools

import jax
import jax.numpy as jnp
from jax.experimental import pallas as pl
from jax.experimental.pallas import tpu as pltpu

_EPS = 1e-5


def _fused_mlp_kernel(x_hbm, W1_ref, g1_ref, be1_ref, W2_ref, g2_ref, be2_ref,
                      Wf1_ref, bf1_ref, Wf2_ref, bf2_ref, out_hbm,
                      h_scr, x_vmem, o_vmem, xsem, osem,
                      *, n_rows, bm):
    nchunks = n_rows // bm
    inv_n = 1.0 / n_rows
    f32 = jnp.float32

    def x_copy(k):
        rows = pl.ds(k * bm, bm)
        return pltpu.make_async_copy(x_hbm.at[rows, :], x_vmem.at[rows, :],
                                     xsem.at[k])

    def o_copy(k):
        rows = pl.ds(k * bm, bm)
        return pltpu.make_async_copy(o_vmem.at[rows, :], out_hbm.at[rows, :],
                                     osem.at[k])

    for k in range(nchunks):
        x_copy(k).start()

    W1 = W1_ref[...]
    S = jnp.zeros((W1.shape[0], W1.shape[0]), f32)
    cs = jnp.zeros((1, W1.shape[0]), f32)
    for k in range(nchunks):
        x_copy(k).wait()
        xk = x_vmem[pl.ds(k * bm, bm), :]
        S = S + jnp.dot(jnp.transpose(xk), xk, preferred_element_type=f32)
        cs = cs + jnp.sum(xk, axis=0, keepdims=True)

    mean1 = jnp.sum(cs.reshape(-1, 1) * W1, axis=0, keepdims=True) * inv_n
    T = jnp.dot(S, W1, preferred_element_type=f32)
    m2 = jnp.sum(W1 * T, axis=0, keepdims=True) * inv_n
    var1 = m2 - mean1 * mean1
    sc1 = g1_ref[...].reshape(1, -1) * jax.lax.rsqrt(var1 + _EPS)
    sh1 = be1_ref[...].reshape(1, -1) - mean1 * sc1
    W1s = W1 * sc1

    W2 = W2_ref[...]
    s = jnp.zeros((1, W2.shape[1]), f32)
    q = jnp.zeros((1, W2.shape[1]), f32)
    for k in range(nchunks):
        rows = pl.ds(k * bm, bm)
        u = jnp.dot(x_vmem[rows, :], W1s, preferred_element_type=f32)
        h1 = jnp.maximum(u + sh1, 0.0)
        h2 = jnp.dot(h1, W2, preferred_element_type=f32)
        h_scr[rows, :] = h2
        s = s + jnp.sum(h2, axis=0, keepdims=True)
        q = q + jnp.sum(h2 * h2, axis=0, keepdims=True)

    mean2 = s * inv_n
    var2 = q * inv_n - mean2 * mean2
    sc2 = g2_ref[...].reshape(1, -1) * jax.lax.rsqrt(var2 + _EPS)
    sh2 = be2_ref[...].reshape(1, -1) - mean2 * sc2
    Wf1s = Wf1_ref[...] * sc2.reshape(-1, 1)
    c = (jnp.sum(sh2.reshape(-1, 1) * Wf1_ref[...], axis=0, keepdims=True)
         + bf1_ref[...].reshape(1, -1))

    Wf2 = Wf2_ref[...]
    out_c = Wf2.shape[1]
    Wf2p = jnp.concatenate(
        [Wf2, jnp.zeros((Wf2.shape[0], 128 - out_c), f32)], axis=1)
    bf2 = bf2_ref[...].reshape(1, -1)
    for k in range(nchunks):
        rows = pl.ds(k * bm, bm)
        m = jnp.dot(h_scr[rows, :], Wf1s, preferred_element_type=f32)
        m = jnp.maximum(m + c, 0.0)
        res = jnp.dot(m, Wf2p, preferred_element_type=f32)
        o_vmem[rows, :] = res[:, :out_c] + bf2
        o_copy(k).start()
    for k in range(nchunks):
        o_copy(k).wait()


def kernel(x, edge_index, edge_attr, W1, b1, g1, be1, W2, b2, g2, be2,
           Wf1, bf1, Wf2, bf2):
    del edge_index, edge_attr, b1, b2
    n, f_in = x.shape
    h_dim = W1.shape[1]
    out_c = Wf2.shape[1]
    bm = 2000

    body = functools.partial(_fused_mlp_kernel, n_rows=n, bm=bm)
    out = pl.pallas_call(
        body,
        in_specs=[pl.BlockSpec(memory_space=pl.MemorySpace.ANY)] + [pl.BlockSpec()] * 10,
        out_specs=pl.BlockSpec(memory_space=pl.MemorySpace.ANY),
        out_shape=jax.ShapeDtypeStruct((n, out_c), jnp.float32),
        compiler_params=pltpu.CompilerParams(
            disable_bounds_checks=True,
            fuse_transposed_lhs_in_matmul=True,
        ),
        scratch_shapes=[
            pltpu.VMEM((n, h_dim), jnp.float32),
            pltpu.VMEM((n, f_in), jnp.float32),
            pltpu.VMEM((n, out_c), jnp.float32),
            pltpu.SemaphoreType.DMA((n // bm,)),
            pltpu.SemaphoreType.DMA((n // bm,)),
        ],
    )(x, W1, g1, be1, W2, g2, be2, Wf1, bf1, Wf2, bf2)
    return out
